# baseline (speedup 1.0000x reference)
import functools

import jax
import jax.numpy as jnp
from jax import lax
from jax.experimental import pallas as pl
from jax.experimental.pallas import tpu as pltpu
from jax.experimental.pallas import tpu_sc as plsc

_B = 1024
_D = 128
_K = 1024
_EPS = 1e-6
_HB = _B // 2
_HK = _K // 2

_NC = 2
_NS = 16
_NW = _NC * _NS
_BPW = _B // _NW


def _rank_chunk(x, w, halfwsq, koff):
    cross = jax.lax.dot_general(
        x, w, (((1,), (0,)), ((), ())),
        precision=jax.lax.Precision.HIGHEST,
        preferred_element_type=jnp.float32,
    )
    r = cross - halfwsq
    m = jnp.max(r, axis=1, keepdims=True)
    kio = jax.lax.broadcasted_iota(jnp.int32, (_HB, _HK), 1) + koff
    i = jnp.min(jnp.where(r == m, kio, _K), axis=1, keepdims=True)
    return m, i


def _som_body(x_hbm, w_hbm, idx_ref, loss_ref, x_v, w_v, sems):
    cx0 = pltpu.make_async_copy(x_hbm.at[pl.ds(0, _HB)], x_v.at[pl.ds(0, _HB)], sems.at[0])
    cw0 = pltpu.make_async_copy(w_hbm.at[:, pl.ds(0, _HK)], w_v.at[:, pl.ds(0, _HK)], sems.at[1])
    cx1 = pltpu.make_async_copy(x_hbm.at[pl.ds(_HB, _HB)], x_v.at[pl.ds(_HB, _HB)], sems.at[2])
    cw1 = pltpu.make_async_copy(w_hbm.at[:, pl.ds(_HK, _HK)], w_v.at[:, pl.ds(_HK, _HK)], sems.at[3])
    cx0.start()
    cw0.start()
    cx1.start()
    cw1.start()

    cx0.wait()
    cw0.wait()
    x0 = x_v[pl.ds(0, _HB), :] + _EPS
    xsq0 = jnp.sum(x0 * x0, axis=1, keepdims=True)
    w0 = w_v[:, pl.ds(0, _HK)]
    hw0 = 0.5 * jnp.sum(w0 * w0, axis=0, keepdims=True)
    m00, i00 = _rank_chunk(x0, w0, hw0, 0)

    cx1.wait()
    x1 = x_v[pl.ds(_HB, _HB), :] + _EPS
    xsq1 = jnp.sum(x1 * x1, axis=1, keepdims=True)
    m10, i10 = _rank_chunk(x1, w0, hw0, 0)

    cw1.wait()
    w1 = w_v[:, pl.ds(_HK, _HK)]
    hw1 = 0.5 * jnp.sum(w1 * w1, axis=0, keepdims=True)
    m01, i01 = _rank_chunk(x0, w1, hw1, _HK)
    m11, i11 = _rank_chunk(x1, w1, hw1, _HK)

    idx_ref[pl.ds(0, _HB), :] = jnp.where(m01 > m00, i01, i00)
    maxv0 = jnp.maximum(m00, m01)
    idx_ref[pl.ds(_HB, _HB), :] = jnp.where(m11 > m10, i11, i10)
    maxv1 = jnp.maximum(m10, m11)

    mind0 = jnp.sqrt(jnp.maximum(xsq0 - 2.0 * maxv0, 0.0))
    mind1 = jnp.sqrt(jnp.maximum(xsq1 - 2.0 * maxv1, 0.0))
    loss_ref[...] = (jnp.sum(mind0, axis=0, keepdims=True)
                     + jnp.sum(mind1, axis=0, keepdims=True)) / _B


def _sc_gather(locf, idxf):
    mesh = plsc.VectorSubcoreMesh(core_axis_name="c", subcore_axis_name="s")

    @functools.partial(
        pl.kernel, mesh=mesh,
        out_type=jax.ShapeDtypeStruct((2 * _B,), jnp.float32),
        compiler_params=pltpu.CompilerParams(needs_layout_passes=False),
        scratch_types=[
            pltpu.VMEM((2 * _K,), jnp.float32),
            pltpu.VMEM((_BPW,), jnp.int32),
            pltpu.VMEM((2 * _BPW,), jnp.float32),
        ],
    )
    def k(loc_hbm, idx_hbm, out_hbm, loc_v, idx_v, out_v):
        wid = lax.axis_index("s") * _NC + lax.axis_index("c")
        base = wid * _BPW
        pltpu.sync_copy(loc_hbm, loc_v)
        pltpu.sync_copy(idx_hbm.at[pl.ds(base, _BPW)], idx_v)
        lane = lax.broadcasted_iota(jnp.int32, (16,), 0)
        for g in range(_BPW // 16):
            ig = idx_v[pl.ds(g * 16, 16)]
            two = ig + ig
            xv = plsc.load_gather(loc_v, [two])
            yv = plsc.load_gather(loc_v, [two + 1])
            pos = lane + lane + g * 32
            plsc.store_scatter(out_v, [pos], xv)
            plsc.store_scatter(out_v, [pos + 1], yv)
        pltpu.sync_copy(out_v, out_hbm.at[pl.ds(2 * base, 2 * _BPW)])

    return k(locf, idxf)


def kernel(input, weight, locations):
    idx, loss = pl.pallas_call(
        _som_body,
        in_specs=[
            pl.BlockSpec(memory_space=pl.ANY),
            pl.BlockSpec(memory_space=pl.ANY),
        ],
        out_shape=(
            jax.ShapeDtypeStruct((_B, 1), jnp.int32),
            jax.ShapeDtypeStruct((1, 1), jnp.float32),
        ),
        scratch_shapes=[
            pltpu.VMEM((_B, _D), jnp.float32),
            pltpu.VMEM((_D, _K), jnp.float32),
            pltpu.SemaphoreType.DMA((8,)),
        ],
    )(input, weight)
    locs = _sc_gather(locations.reshape(2 * _K), idx.reshape(_B))
    return locs.reshape(_B, 1, 2), loss[0, 0]


# speedup vs baseline: 2.8698x; 2.8698x over previous
import jax
import jax.numpy as jnp
from jax.experimental import pallas as pl
from jax.experimental.pallas import tpu as pltpu

_B = 1024
_D = 128
_K = 1024
_EPS = 1e-6
_RB = 1024
_NRB = _B // _RB
_KC = 1024
_NKC = _K // _KC
_LW = 128
_NSUB = _KC // _LW
_G = 32
_GW = _K // _G


def _gather_locs(idx, tblb):
    hi = idx >> 5
    lo = idx & (_GW - 1)
    giota = jax.lax.broadcasted_iota(jnp.int32, (_RB, _G), 1)
    ohhi = (giota == hi).astype(jnp.bfloat16)
    pairs = jax.lax.dot_general(
        ohhi, tblb, (((1,), (0,)), ((), ())),
        preferred_element_type=jnp.float32)
    piota = jax.lax.broadcasted_iota(jnp.int32, (_RB, 2 * _GW), 1)
    sel = ((piota >> 1) == lo).astype(jnp.float32)
    par = (piota & 1).astype(jnp.float32)
    prod = pairs * sel
    locx = jnp.sum(prod * (1.0 - par), axis=1, keepdims=True)
    locy = jnp.sum(prod * par, axis=1, keepdims=True)
    return jnp.concatenate([locx, locy], axis=1)


def _som_body(x_hbm, w_hbm, tbl_hbm, locs_ref, loss_ref,
              x_v, w_v, tbl_v, hw_v, sems):
    cxs = [pltpu.make_async_copy(x_hbm.at[pl.ds(rb * _RB, _RB)],
                                 x_v.at[pl.ds(rb * _RB, _RB)], sems.at[rb])
           for rb in range(_NRB)]
    cw0 = pltpu.make_async_copy(w_hbm.at[:, pl.ds(0, _K // 2)],
                                w_v.at[:, pl.ds(0, _K // 2)], sems.at[4])
    cw1 = pltpu.make_async_copy(w_hbm.at[:, pl.ds(_K // 2, _K // 2)],
                                w_v.at[:, pl.ds(_K // 2, _K // 2)], sems.at[5])
    ctbl = pltpu.make_async_copy(tbl_hbm, tbl_v, sems.at[6])
    cxs[0].start()
    cw0.start()
    for c in cxs[1:]:
        c.start()
    cw1.start()
    ctbl.start()

    lane = jax.lax.broadcasted_iota(jnp.int32, (_RB, _LW), 1)
    loss_acc = None
    for rb in range(_NRB):
        cxs[rb].wait()
        if rb == 0:
            cw0.wait()
        x = x_v[pl.ds(rb * _RB, _RB), :] + _EPS
        xsq = jnp.sum(x * x, axis=1, keepdims=True)
        M = None
        I = None
        for c in range(_NKC):
            if rb == 0 and c == _NKC // 2:
                cw1.wait()
            wc = w_v[:, pl.ds(c * _KC, _KC)]
            if rb == 0:
                hwc = 0.5 * jnp.sum(wc * wc, axis=0, keepdims=True)
                hw_v[:, pl.ds(c * _KC, _KC)] = hwc
            else:
                hwc = hw_v[:, pl.ds(c * _KC, _KC)]
            cross = jax.lax.dot_general(
                x, wc, (((1,), (0,)), ((), ())),
                precision=jax.lax.Precision.HIGHEST,
                preferred_element_type=jnp.float32,
            )
            for s in range(_NSUB):
                sub = c * _NSUB + s
                crs = (jax.lax.slice_in_dim(cross, s * _LW, (s + 1) * _LW, axis=1)
                       if _NSUB > 1 else cross)
                r = crs - jax.lax.slice_in_dim(hwc, s * _LW, (s + 1) * _LW, axis=1)
                if sub == 0:
                    M = r
                    I = jnp.zeros((_RB, _LW), jnp.int32)
                else:
                    upd = r > M
                    I = jnp.where(upd, sub, I)
                    M = jnp.maximum(M, r)
        rowmax = jnp.max(M, axis=1, keepdims=True)
        kfull = I * _LW + lane
        idx = jnp.min(jnp.where(M == rowmax, kfull, _K), axis=1, keepdims=True)
        if rb == 0:
            ctbl.wait()
            tblb = tbl_v[...].astype(jnp.bfloat16)
        locs_ref[pl.ds(rb * _RB, _RB), :] = _gather_locs(idx, tblb)
        mind = jnp.sqrt(jnp.maximum(xsq - 2.0 * rowmax, 0.0))
        part = jnp.sum(mind, axis=0, keepdims=True)
        loss_acc = part if rb == 0 else loss_acc + part
    loss_ref[...] = loss_acc / _B


def kernel(input, weight, locations):
    locs, loss = pl.pallas_call(
        _som_body,
        in_specs=[
            pl.BlockSpec(memory_space=pl.ANY),
            pl.BlockSpec(memory_space=pl.ANY),
            pl.BlockSpec(memory_space=pl.ANY),
        ],
        out_shape=(
            jax.ShapeDtypeStruct((_B, 2), jnp.float32),
            jax.ShapeDtypeStruct((1, 1), jnp.float32),
        ),
        scratch_shapes=[
            pltpu.VMEM((_B, _D), jnp.float32),
            pltpu.VMEM((_D, _K), jnp.float32),
            pltpu.VMEM((_G, 2 * _GW), jnp.float32),
            pltpu.VMEM((1, _K), jnp.float32),
            pltpu.SemaphoreType.DMA((8,)),
        ],
    )(input, weight, locations.reshape(_G, 2 * _GW))
    return locs.reshape(_B, 1, 2), loss[0, 0]


# speedup vs baseline: 2.9449x; 1.0262x over previous
import jax
import jax.numpy as jnp
from jax.experimental import pallas as pl
from jax.experimental.pallas import tpu as pltpu

_B = 1024
_D = 128
_K = 1024
_EPS = 1e-6
_RB = 1024
_NRB = _B // _RB
_KC = 1024
_NKC = _K // _KC
_LW = 128
_NSUB = _KC // _LW
_G = 32
_GW = _K // _G


def _gather_locs(idx, tblb):
    hi = idx >> 5
    lo = idx & (_GW - 1)
    giota = jax.lax.broadcasted_iota(jnp.int32, (_RB, _G), 1)
    ohhi = (giota == hi).astype(jnp.bfloat16)
    pairs = jax.lax.dot_general(
        ohhi, tblb, (((1,), (0,)), ((), ())),
        preferred_element_type=jnp.float32)
    piota = jax.lax.broadcasted_iota(jnp.int32, (_RB, 2 * _GW), 1)
    sel = ((piota >> 1) == lo).astype(jnp.float32)
    par = (piota & 1).astype(jnp.float32)
    prod = pairs * sel
    locx = jnp.sum(prod * (1.0 - par), axis=1, keepdims=True)
    locy = jnp.sum(prod * par, axis=1, keepdims=True)
    return jnp.concatenate([locx, locy], axis=1)


def _som_body(x_v, w_v, tbl_v, locs_ref, loss_ref, hw_v):

    lane = jax.lax.broadcasted_iota(jnp.int32, (_RB, _LW), 1)
    loss_acc = None
    for rb in range(_NRB):
        x = x_v[pl.ds(rb * _RB, _RB), :] + _EPS
        xsq = jnp.sum(x * x, axis=1, keepdims=True)
        M = None
        I = None
        for c in range(_NKC):
            wc = w_v[:, pl.ds(c * _KC, _KC)]
            if rb == 0:
                hwc = 0.5 * jnp.sum(wc * wc, axis=0, keepdims=True)
                hw_v[:, pl.ds(c * _KC, _KC)] = hwc
            else:
                hwc = hw_v[:, pl.ds(c * _KC, _KC)]
            cross = jax.lax.dot_general(
                x, wc, (((1,), (0,)), ((), ())),
                precision=jax.lax.Precision.HIGHEST,
                preferred_element_type=jnp.float32,
            )
            for s in range(_NSUB):
                sub = c * _NSUB + s
                crs = (jax.lax.slice_in_dim(cross, s * _LW, (s + 1) * _LW, axis=1)
                       if _NSUB > 1 else cross)
                r = crs - jax.lax.slice_in_dim(hwc, s * _LW, (s + 1) * _LW, axis=1)
                if sub == 0:
                    M = r
                    I = jnp.zeros((_RB, _LW), jnp.int32)
                else:
                    upd = r > M
                    I = jnp.where(upd, sub, I)
                    M = jnp.maximum(M, r)
        rowmax = jnp.max(M, axis=1, keepdims=True)
        kfull = I * _LW + lane
        idx = jnp.min(jnp.where(M == rowmax, kfull, _K), axis=1, keepdims=True)
        if rb == 0:
            tblb = tbl_v[...].astype(jnp.bfloat16)
        locs_ref[pl.ds(rb * _RB, _RB), :] = _gather_locs(idx, tblb)
        mind = jnp.sqrt(jnp.maximum(xsq - 2.0 * rowmax, 0.0))
        part = jnp.sum(mind, axis=0, keepdims=True)
        loss_acc = part if rb == 0 else loss_acc + part
    loss_ref[...] = loss_acc / _B


def kernel(input, weight, locations):
    locs, loss = pl.pallas_call(
        _som_body,
        out_shape=(
            jax.ShapeDtypeStruct((_B, 2), jnp.float32),
            jax.ShapeDtypeStruct((1, 1), jnp.float32),
        ),
        scratch_shapes=[
            pltpu.VMEM((1, _K), jnp.float32),
        ],
    )(input, weight, locations.reshape(_G, 2 * _GW))
    return locs.reshape(_B, 1, 2), loss[0, 0]


# speedup vs baseline: 2.9454x; 1.0002x over previous
import jax
import jax.numpy as jnp
from jax.experimental import pallas as pl

_B = 1024
_D = 128
_K = 1024
_EPS = 1e-6
_RB = 1024
_NRB = _B // _RB
_KC = 1024
_NKC = _K // _KC
_LW = 128
_NSUB = _KC // _LW
_G = 32
_GW = _K // _G


def _gather_locs(idx, tblb):
    hi = idx >> 5
    lo = idx & (_GW - 1)
    giota = jax.lax.broadcasted_iota(jnp.int32, (_RB, _G), 1)
    ohhi = (giota == hi).astype(jnp.bfloat16)
    pairs = jax.lax.dot_general(
        ohhi, tblb, (((1,), (0,)), ((), ())),
        preferred_element_type=jnp.float32)
    piota = jax.lax.broadcasted_iota(jnp.int32, (_RB, 2 * _GW), 1)
    sel = ((piota >> 1) == lo).astype(jnp.float32)
    par = (piota & 1).astype(jnp.float32)
    prod = pairs * sel
    locx = jnp.sum(prod * (1.0 - par), axis=1, keepdims=True)
    locy = jnp.sum(prod * par, axis=1, keepdims=True)
    return jnp.concatenate([locx, locy], axis=1)


def _som_body(x_v, w_v, tbl_v, locs_ref, loss_ref):

    lane = jax.lax.broadcasted_iota(jnp.int32, (_RB, _LW), 1)
    loss_acc = None
    for rb in range(_NRB):
        x = x_v[pl.ds(rb * _RB, _RB), :] + _EPS
        xsq = jnp.sum(x * x, axis=1, keepdims=True)
        M = None
        I = None
        for c in range(_NKC):
            wc = w_v[:, pl.ds(c * _KC, _KC)]
            hwc = 0.5 * jnp.sum(wc * wc, axis=0, keepdims=True)
            cross = jax.lax.dot_general(
                x, wc, (((1,), (0,)), ((), ())),
                precision=jax.lax.Precision.HIGHEST,
                preferred_element_type=jnp.float32,
            )
            for s in range(_NSUB):
                sub = c * _NSUB + s
                crs = (jax.lax.slice_in_dim(cross, s * _LW, (s + 1) * _LW, axis=1)
                       if _NSUB > 1 else cross)
                r = crs - jax.lax.slice_in_dim(hwc, s * _LW, (s + 1) * _LW, axis=1)
                if sub == 0:
                    M = r
                    I = jnp.zeros((_RB, _LW), jnp.int32)
                else:
                    upd = r > M
                    I = jnp.where(upd, sub, I)
                    M = jnp.maximum(M, r)
        rowmax = jnp.max(M, axis=1, keepdims=True)
        kfull = I * _LW + lane
        idx = jnp.min(jnp.where(M == rowmax, kfull, _K), axis=1, keepdims=True)
        if rb == 0:
            tblb = tbl_v[...].astype(jnp.bfloat16)
        locs_ref[pl.ds(rb * _RB, _RB), :] = _gather_locs(idx, tblb)
        mind = jnp.sqrt(jnp.maximum(xsq - 2.0 * rowmax, 0.0))
        part = jnp.sum(mind, axis=0, keepdims=True)
        loss_acc = part if rb == 0 else loss_acc + part
    loss_ref[...] = loss_acc / _B


def kernel(input, weight, locations):
    locs, loss = pl.pallas_call(
        _som_body,
        out_shape=(
            jax.ShapeDtypeStruct((_B, 2), jnp.float32),
            jax.ShapeDtypeStruct((1, 1), jnp.float32),
        ),
    )(input, weight, locations.reshape(_G, 2 * _GW))
    return locs.reshape(_B, 1, 2), loss[0, 0]
